# baseline (speedup 1.0000x reference)
import dataclasses
import functools

import jax
import jax.numpy as jnp
from jax import lax
from jax.experimental import pallas as pl
from jax.experimental.pallas import tpu as pltpu
from jax.experimental.pallas import tpu_sc as plsc

_BETA = 0.5
_C_CONST = 1.0
_NEG_SLOPE = 0.2

_NC = 2
_NS = 16
_L = 16
_DP = 144


def _lrelu(t):
    return jnp.where(t >= 0.0, t, _NEG_SLOPE * t)


def kernel(x, edge_index, W, att_src, att_dst, bias):
    N, D_IN = x.shape
    D = att_src.shape[-1]
    E = edge_index.shape[1]
    src = edge_index[0].astype(jnp.int32)
    dst = edge_index[1].astype(jnp.int32)

    def prep_body(x_ref, w_ref, asrc_ref, adst_ref,
                  xpp_ref, adc_ref, ws_ref):
        xp = jnp.dot(x_ref[...], w_ref[...], preferred_element_type=jnp.float32)
        a_s = jnp.dot(xp, asrc_ref[...].T, preferred_element_type=jnp.float32)
        a_d = jnp.dot(xp, adst_ref[...].T, preferred_element_type=jnp.float32)
        m = jnp.max(a_s)
        c = _lrelu(m + a_d)
        ws_ref[...] = jnp.exp(_lrelu(a_s + a_d) - c)
        z1 = jnp.zeros((xp.shape[0], 1), jnp.float32)
        z14 = jnp.zeros((xp.shape[0], _DP - D - 2), jnp.float32)
        xpp_ref[...] = jnp.concatenate([xp, z1, a_s, z14], axis=1)
        adc_ref[...] = jnp.concatenate(
            [a_d, c, jnp.zeros((xp.shape[0], _L - 2), jnp.float32)], axis=1)

    xpp, adc, wself = pl.pallas_call(
        prep_body,
        out_shape=[
            jax.ShapeDtypeStruct((N, _DP), jnp.float32),
            jax.ShapeDtypeStruct((N, _L), jnp.float32),
            jax.ShapeDtypeStruct((N, 1), jnp.float32),
        ],
    )(x, W, att_src, att_dst)

    TILES = _NC * _NS
    EPT = E // TILES
    CH = 80
    NCHUNK = EPT // CH
    G = CH // _L
    RPT = -(-N // (_NS * CH)) * CH
    N_PAD = _NS * RPT

    mesh = plsc.VectorSubcoreMesh(core_axis_name="c", subcore_axis_name="s")
    sc_params = pltpu.CompilerParams()
    if "needs_layout_passes" in pltpu.CompilerParams.__dataclass_fields__:
        sc_params = dataclasses.replace(sc_params, needs_layout_passes=False)
    if "use_tc_tiling_on_sc" in pltpu.CompilerParams.__dataclass_fields__:
        sc_params = dataclasses.replace(sc_params, use_tc_tiling_on_sc=False)

    @functools.partial(
        pl.kernel,
        mesh=mesh,
        compiler_params=sc_params,
        out_type=jax.ShapeDtypeStruct((_NC, N_PAD, _DP), jnp.float32),
        scratch_types=[
            pltpu.VMEM((2, 2, CH), jnp.int32),
            pltpu.VMEM((2, CH), jnp.int32),
            pltpu.VMEM((2, CH, _L), jnp.float32),
            pltpu.VMEM((2, CH, _DP), jnp.float32),
            pltpu.VMEM_SHARED((N_PAD, _DP), jnp.float32),
            pltpu.SemaphoreType.DMA((2,)),
            pltpu.SemaphoreType.DMA((2,)),
            pltpu.SemaphoreType.DMA((2,)),
            pltpu.SemaphoreType.DMA((2,)),
        ],
    )
    def edge_kernel(ei_hbm, adc_hbm, xpp_hbm, oacc_hbm,
                    eidx_v, didx_s, adc_v, rows_v, acc,
                    sem_ei, sem_row, sem_adc, sem_sc):
        cid = lax.axis_index("c")
        sid = lax.axis_index("s")
        base_c = (cid * _NS + sid) * NCHUNK
        base_r = sid * RPT

        zv = jnp.zeros((_L,), jnp.float32)

        @pl.loop(0, CH)
        def _zero_rows(i):
            for j in range(_DP // _L):
                rows_v[0, i, pl.ds(j * _L, _L)] = zv

        @pl.loop(0, RPT, step=CH)
        def _zero_acc(r):
            pltpu.sync_copy(rows_v.at[0], acc.at[pl.ds(base_r + r, CH)])

        plsc.subcore_barrier()

        lane = lax.iota(jnp.int32, _L)
        col_w = lane * 0 + D

        def issue_idx(k, b):
            pltpu.async_copy(ei_hbm.at[base_c + k], eidx_v.at[b],
                             sem_ei.at[b])

        def wait_idx(k, b):
            pltpu.make_async_copy(ei_hbm.at[base_c + k], eidx_v.at[b],
                                  sem_ei.at[b]).wait()

        def issue_gather(b):
            pltpu.async_copy(xpp_hbm.at[eidx_v.at[b, 0]], rows_v.at[b],
                             sem_row.at[b])
            pltpu.async_copy(adc_hbm.at[eidx_v.at[b, 1]], adc_v.at[b],
                             sem_adc.at[b])

        def wait_gather(b):
            pltpu.make_async_copy(xpp_hbm.at[eidx_v.at[b, 0]], rows_v.at[b],
                                  sem_row.at[b]).wait()
            pltpu.make_async_copy(adc_hbm.at[eidx_v.at[b, 1]], adc_v.at[b],
                                  sem_adc.at[b]).wait()

        def issue_scatter(b):
            pltpu.async_copy(rows_v.at[b], acc.at[didx_s.at[b]],
                             sem_sc.at[b], add=True)

        def wait_scatter(b):
            pltpu.make_async_copy(rows_v.at[b], acc.at[didx_s.at[b]],
                                  sem_sc.at[b]).wait()

        issue_idx(0, 0)
        wait_idx(0, 0)
        issue_gather(0)
        issue_idx(1, 1)

        def chunk_body(k, b):
            nb = 1 - b

            @pl.when(k < NCHUNK - 1)
            def _():
                wait_idx(k + 1, nb)

                @pl.when(k >= 1)
                def _():
                    wait_scatter(nb)

                issue_gather(nb)

            @pl.when(k == NCHUNK - 1)
            def _():
                wait_scatter(nb)

            wait_gather(b)

            @plsc.parallel_loop(0, G, unroll=2)
            def _grp(g):
                rows16 = g * _L + lane
                asv = plsc.load_gather(rows_v.at[b], [rows16, lane * 0 + D + 1])
                adv = plsc.load_gather(adc_v.at[b], [rows16, lane * 0])
                cv = plsc.load_gather(adc_v.at[b], [rows16, lane * 0 + 1])
                t = asv + adv
                wv = jnp.exp(jnp.where(t >= 0.0, t, _NEG_SLOPE * t) - cv)
                plsc.store_scatter(rows_v.at[b], [rows16, col_w], wv)

            @plsc.parallel_loop(0, CH, unroll=4)
            def _scale(e):
                ws = rows_v[b, e, pl.ds(D, _L)][0]
                for j in range(D // _L):
                    sl = pl.ds(j * _L, _L)
                    rows_v[b, e, sl] = rows_v[b, e, sl] * ws

            for i in range(G):
                sl = pl.ds(i * _L, _L)
                didx_s[b, sl] = eidx_v[b, 1, sl]

            issue_scatter(b)

            @pl.when(k < NCHUNK - 2)
            def _():
                issue_idx(k + 2, b)

        @pl.loop(0, NCHUNK - 1, step=2)
        def _pair(k):
            chunk_body(k, 0)
            chunk_body(k + 1, 1)

        chunk_body(NCHUNK - 1, (NCHUNK - 1) % 2)
        wait_scatter((NCHUNK - 1) % 2)

        plsc.subcore_barrier()
        pltpu.sync_copy(acc.at[pl.ds(base_r, RPT)],
                        oacc_hbm.at[cid, pl.ds(base_r, RPT)])

    eidx = jnp.stack(
        [src.reshape(E // CH, CH), dst.reshape(E // CH, CH)], axis=1)
    oacc = edge_kernel(eidx, adc, xpp)

    def fin_body(a_ref, xpp_ref, ws_ref, b_ref, o_ref):
        a0 = a_ref[0][:N]
        a1 = a_ref[1][:N]
        xp = xpp_ref[:, :D]
        num = a0[:, :D] + a1[:, :D] + ws_ref[...] * xp
        den = a0[:, D:D + 1] + a1[:, D:D + 1] + ws_ref[...]
        z = num / den + b_ref[...]
        elu = jnp.where(z > 0.0, z, jnp.exp(z) - 1.0)
        o_ref[...] = _BETA * z + (_C_CONST - _BETA) * elu

    out = pl.pallas_call(
        fin_body,
        out_shape=jax.ShapeDtypeStruct((N, D), jnp.float32),
    )(oacc, xpp, wself, bias.reshape(1, D))

    return out


# speedup vs baseline: 1.0702x; 1.0702x over previous
import dataclasses
import functools

import jax
import jax.numpy as jnp
from jax import lax
from jax.experimental import pallas as pl
from jax.experimental.pallas import tpu as pltpu
from jax.experimental.pallas import tpu_sc as plsc

_BETA = 0.5
_C_CONST = 1.0
_NEG_SLOPE = 0.2

_NC = 2
_NS = 16
_L = 16
_DP = 144


def _lrelu(t):
    return jnp.where(t >= 0.0, t, _NEG_SLOPE * t)


def kernel(x, edge_index, W, att_src, att_dst, bias):
    N, D_IN = x.shape
    D = att_src.shape[-1]
    E = edge_index.shape[1]
    src = edge_index[0].astype(jnp.int32)
    dst = edge_index[1].astype(jnp.int32)

    def prep_body(x_ref, w_ref, asrc_ref, adst_ref,
                  xpp_ref, adc_ref, ws_ref):
        xp = jnp.dot(x_ref[...], w_ref[...], preferred_element_type=jnp.float32)
        a_s = jnp.dot(xp, asrc_ref[...].T, preferred_element_type=jnp.float32)
        a_d = jnp.dot(xp, adst_ref[...].T, preferred_element_type=jnp.float32)
        m = jnp.max(a_s)
        c = _lrelu(m + a_d)
        ws_ref[...] = jnp.exp(_lrelu(a_s + a_d) - c)
        z1 = jnp.zeros((xp.shape[0], 1), jnp.float32)
        z14 = jnp.zeros((xp.shape[0], _DP - D - 2), jnp.float32)
        xpp_ref[...] = jnp.concatenate([xp, z1, a_s, z14], axis=1)
        adc_ref[...] = jnp.concatenate(
            [a_d, c, jnp.zeros((xp.shape[0], _L - 2), jnp.float32)], axis=1)

    xpp, adc, wself = pl.pallas_call(
        prep_body,
        out_shape=[
            jax.ShapeDtypeStruct((N, _DP), jnp.float32),
            jax.ShapeDtypeStruct((N, _L), jnp.float32),
            jax.ShapeDtypeStruct((N, 1), jnp.float32),
        ],
    )(x, W, att_src, att_dst)

    TILES = _NC * _NS
    EPT = E // TILES
    CH = 80
    NCHUNK = EPT // CH
    G = CH // _L
    RPT = -(-N // (_NS * CH)) * CH
    N_PAD = _NS * RPT

    mesh = plsc.VectorSubcoreMesh(core_axis_name="c", subcore_axis_name="s")
    sc_params = pltpu.CompilerParams()
    if "needs_layout_passes" in pltpu.CompilerParams.__dataclass_fields__:
        sc_params = dataclasses.replace(sc_params, needs_layout_passes=False)
    if "use_tc_tiling_on_sc" in pltpu.CompilerParams.__dataclass_fields__:
        sc_params = dataclasses.replace(sc_params, use_tc_tiling_on_sc=False)

    @functools.partial(
        pl.kernel,
        mesh=mesh,
        compiler_params=sc_params,
        out_type=jax.ShapeDtypeStruct((_NC, N_PAD, _DP), jnp.float32),
        scratch_types=[
            pltpu.VMEM((2, CH), jnp.int32),
            pltpu.VMEM((2, CH), jnp.int32),
            pltpu.VMEM((2, CH), jnp.int32),
            pltpu.VMEM((2, CH, _L), jnp.float32),
            pltpu.VMEM((2, CH, _DP), jnp.float32),
            pltpu.VMEM_SHARED((N_PAD, _DP), jnp.float32),
            pltpu.SemaphoreType.DMA((2,)),
            pltpu.SemaphoreType.DMA((2,)),
            pltpu.SemaphoreType.DMA((2,)),
            pltpu.SemaphoreType.DMA((2,)),
            pltpu.SemaphoreType.DMA((2,)),
        ],
    )
    def edge_kernel(src_hbm, dst_hbm, adc_hbm, xpp_hbm, oacc_hbm,
                    sidx_v, didx_v, didx_s, adc_v, rows_v, acc,
                    sem_si, sem_di, sem_row, sem_adc, sem_sc):
        cid = lax.axis_index("c")
        sid = lax.axis_index("s")
        base_c = (cid * _NS + sid) * NCHUNK
        base_r = sid * RPT

        zv = jnp.zeros((_L,), jnp.float32)

        @pl.loop(0, CH)
        def _zero_rows(i):
            for j in range(_DP // _L):
                rows_v[0, i, pl.ds(j * _L, _L)] = zv

        @pl.loop(0, RPT, step=CH)
        def _zero_acc(r):
            pltpu.sync_copy(rows_v.at[0], acc.at[pl.ds(base_r + r, CH)])

        plsc.subcore_barrier()

        lane = lax.iota(jnp.int32, _L)
        col_w = lane * 0 + D

        def issue_idx(k, b):
            pltpu.async_copy(src_hbm.at[base_c + k], sidx_v.at[b],
                             sem_si.at[b])
            pltpu.async_copy(dst_hbm.at[base_c + k], didx_v.at[b],
                             sem_di.at[b])

        def wait_idx(k, b):
            pltpu.make_async_copy(src_hbm.at[base_c + k], sidx_v.at[b],
                                  sem_si.at[b]).wait()
            pltpu.make_async_copy(dst_hbm.at[base_c + k], didx_v.at[b],
                                  sem_di.at[b]).wait()

        def issue_gather(b):
            pltpu.async_copy(xpp_hbm.at[sidx_v.at[b]], rows_v.at[b],
                             sem_row.at[b])
            pltpu.async_copy(adc_hbm.at[didx_v.at[b]], adc_v.at[b],
                             sem_adc.at[b])

        def wait_gather(b):
            pltpu.make_async_copy(xpp_hbm.at[sidx_v.at[b]], rows_v.at[b],
                                  sem_row.at[b]).wait()
            pltpu.make_async_copy(adc_hbm.at[didx_v.at[b]], adc_v.at[b],
                                  sem_adc.at[b]).wait()

        def issue_scatter(b):
            pltpu.async_copy(rows_v.at[b], acc.at[didx_s.at[b]],
                             sem_sc.at[b], add=True)

        def wait_scatter(b):
            pltpu.make_async_copy(rows_v.at[b], acc.at[didx_s.at[b]],
                                  sem_sc.at[b]).wait()

        issue_idx(0, 0)
        wait_idx(0, 0)
        issue_gather(0)
        issue_idx(1, 1)

        def chunk_body(k, b):
            nb = 1 - b

            @pl.when(k < NCHUNK - 1)
            def _():
                wait_idx(k + 1, nb)

                @pl.when(k >= 1)
                def _():
                    wait_scatter(nb)

                issue_gather(nb)

            @pl.when(k == NCHUNK - 1)
            def _():
                wait_scatter(nb)

            wait_gather(b)

            @plsc.parallel_loop(0, G, unroll=2)
            def _grp(g):
                rows16 = g * _L + lane
                asv = plsc.load_gather(rows_v.at[b], [rows16, lane * 0 + D + 1])
                adv = plsc.load_gather(adc_v.at[b], [rows16, lane * 0])
                cv = plsc.load_gather(adc_v.at[b], [rows16, lane * 0 + 1])
                t = asv + adv
                wv = jnp.exp(jnp.where(t >= 0.0, t, _NEG_SLOPE * t) - cv)
                plsc.store_scatter(rows_v.at[b], [rows16, col_w], wv)

            @plsc.parallel_loop(0, CH, unroll=4)
            def _scale(e):
                ws = rows_v[b, e, pl.ds(D, _L)][0]
                for j in range(D // _L):
                    sl = pl.ds(j * _L, _L)
                    rows_v[b, e, sl] = rows_v[b, e, sl] * ws

            for i in range(G):
                sl = pl.ds(i * _L, _L)
                didx_s[b, sl] = didx_v[b, sl]

            issue_scatter(b)

            @pl.when(k < NCHUNK - 2)
            def _():
                issue_idx(k + 2, b)

        @pl.loop(0, NCHUNK - 1, step=2)
        def _pair(k):
            chunk_body(k, 0)
            chunk_body(k + 1, 1)

        chunk_body(NCHUNK - 1, (NCHUNK - 1) % 2)
        wait_scatter((NCHUNK - 1) % 2)

        plsc.subcore_barrier()
        pltpu.sync_copy(acc.at[pl.ds(base_r, RPT)],
                        oacc_hbm.at[cid, pl.ds(base_r, RPT)])

    oacc = edge_kernel(src.reshape(E // CH, CH), dst.reshape(E // CH, CH),
                       adc, xpp)

    def fin_body(a_ref, xpp_ref, ws_ref, b_ref, o_ref):
        a0 = a_ref[0][:N]
        a1 = a_ref[1][:N]
        xp = xpp_ref[:, :D]
        num = a0[:, :D] + a1[:, :D] + ws_ref[...] * xp
        den = a0[:, D:D + 1] + a1[:, D:D + 1] + ws_ref[...]
        z = num / den + b_ref[...]
        elu = jnp.where(z > 0.0, z, jnp.exp(z) - 1.0)
        o_ref[...] = _BETA * z + (_C_CONST - _BETA) * elu

    out = pl.pallas_call(
        fin_body,
        out_shape=jax.ShapeDtypeStruct((N, D), jnp.float32),
    )(oacc, xpp, wself, bias.reshape(1, D))

    return out
